# baseline (speedup 1.0000x reference)
import functools

import jax
import jax.numpy as jnp
from jax import lax
from jax.experimental import pallas as pl
from jax.experimental.pallas import tpu as pltpu
from jax.experimental.pallas import tpu_sc as plsc

_BLK = 256
_CHK = 2048


def _tc_body(n_rows, n_steps, x_ref, w_ref, idx_ref, loss_ref, perp_ref,
             counts_ref, lsum_ref):
    i = pl.program_id(0)
    k = w_ref.shape[0]
    d_dim = w_ref.shape[1]

    @pl.when(i == 0)
    def _init():
        counts_ref[...] = jnp.zeros_like(counts_ref)
        lsum_ref[...] = jnp.zeros_like(lsum_ref)

    x = x_ref[...]
    w = w_ref[...]
    sx = jnp.sum(x * x, axis=1)
    sw = jnp.sum(w * w, axis=1)
    xb = x.astype(jnp.bfloat16)

    rv = jnp.full((x.shape[0],), jnp.inf, jnp.float32)
    idx = jnp.zeros((x.shape[0],), jnp.int32)
    for c in range(k // _CHK):
        wc = w_ref[pl.ds(c * _CHK, _CHK), :]
        m = lax.dot_general(xb, wc, (((1,), (1,)), ((), ())),
                            preferred_element_type=jnp.float32)
        chunk = sx[:, None] + sw[c * _CHK:(c + 1) * _CHK][None, :] - 2.0 * m
        mc = jnp.min(chunk, axis=1)
        ic = jnp.argmin(chunk, axis=1).astype(jnp.int32) + c * _CHK
        take = (mc < rv) | ((mc == rv) & (ic < idx))
        rv = jnp.where(take, mc, rv).astype(jnp.bfloat16).astype(jnp.float32)
        idx = jnp.where(take, ic, idx)
    idx_ref[0, 0, :] = idx
    lsum_ref[...] += jnp.sum(rv).reshape(1, 1)
    cols = lax.broadcasted_iota(jnp.int32, (idx.shape[0], k), 1)
    onehot = jnp.where(idx[:, None] == cols, 1.0, 0.0)
    counts_ref[...] += jnp.sum(onehot, axis=0)[None, :]

    @pl.when(i == n_steps - 1)
    def _fin():
        avg = counts_ref[...] * (1.0 / n_rows)
        ent = jnp.sum(avg * jnp.log(avg + 1e-10))
        perp_ref[...] = jnp.exp(-ent).reshape(1, 1)
        loss_ref[...] = lsum_ref[...] * (0.25 / (n_rows * d_dim))


def _tc_stage(x, w):
    n, d_dim = x.shape
    k = w.shape[0]
    steps = n // _BLK
    return pl.pallas_call(
        functools.partial(_tc_body, n, steps),
        grid=(steps,),
        in_specs=[
            pl.BlockSpec((_BLK, d_dim), lambda i: (i, 0)),
            pl.BlockSpec((k, d_dim), lambda i: (0, 0)),
        ],
        out_specs=[
            pl.BlockSpec((1, 1, _BLK), lambda i: (i, 0, 0)),
            pl.BlockSpec((1, 1), lambda i: (0, 0)),
            pl.BlockSpec((1, 1), lambda i: (0, 0)),
        ],
        out_shape=[
            jax.ShapeDtypeStruct((steps, 1, _BLK), jnp.int32),
            jax.ShapeDtypeStruct((1, 1), jnp.float32),
            jax.ShapeDtypeStruct((1, 1), jnp.float32),
        ],
        scratch_shapes=[
            pltpu.VMEM((1, k), jnp.float32),
            pltpu.VMEM((1, 1), jnp.float32),
        ],
    )(x, w)


def _sc_gather(w, idx):
    n = idx.shape[0]
    d_dim = w.shape[1]
    info = plsc.get_sparse_core_info()
    nw = info.num_cores * info.num_subcores
    bpw = n // nw
    mesh = plsc.VectorSubcoreMesh(core_axis_name="c", subcore_axis_name="s")

    @functools.partial(
        pl.kernel,
        mesh=mesh,
        compiler_params=pltpu.CompilerParams(use_tc_tiling_on_sc=False),
        out_type=jax.ShapeDtypeStruct((n, d_dim), jnp.float32),
        scratch_types=[
            pltpu.VMEM((bpw,), jnp.int32),
            pltpu.VMEM((bpw, d_dim), jnp.float32),
            pltpu.SemaphoreType.DMA,
        ],
    )
    def gather_kernel(w_hbm, idx_hbm, out_hbm, idx_v, rows_v, sem):
        wid = lax.axis_index("s") * info.num_cores + lax.axis_index("c")
        base = wid * bpw
        pltpu.sync_copy(idx_hbm.at[pl.ds(base, bpw)], idx_v)
        pltpu.async_copy(w_hbm.at[idx_v], rows_v, sem).wait()
        pltpu.sync_copy(rows_v, out_hbm.at[pl.ds(base, bpw)])

    return gather_kernel(w, idx)


def kernel(inputs, w):
    d_dim = inputs.shape[-1]
    x = inputs.reshape(-1, d_dim)
    idx3, loss, perp = _tc_stage(x, w)
    idx = idx3.reshape(-1)
    quantized = _sc_gather(w, idx).reshape(inputs.shape)
    quantized_st = inputs + (quantized - inputs)
    return (loss[0, 0], quantized_st, perp[0, 0], idx[:, None])


# speedup vs baseline: 1.1020x; 1.1020x over previous
import functools

import jax
import jax.numpy as jnp
from jax import lax
from jax.experimental import pallas as pl
from jax.experimental.pallas import tpu as pltpu
from jax.experimental.pallas import tpu_sc as plsc

_BLK = 256
_CHK = 2048


def _tc_body(n_steps, x_ref, w_ref, idx_ref, lsum_ref, acc_ref):
    i = pl.program_id(0)
    k = w_ref.shape[0]

    @pl.when(i == 0)
    def _init():
        acc_ref[...] = jnp.zeros_like(acc_ref)

    x = x_ref[...]
    w = w_ref[...]
    sx = jnp.sum(x * x, axis=1)
    sw = jnp.sum(w * w, axis=1)
    xb = x.astype(jnp.bfloat16)

    rv = jnp.full((x.shape[0],), jnp.inf, jnp.float32)
    idx = jnp.zeros((x.shape[0],), jnp.int32)
    for c in range(k // _CHK):
        wc = w_ref[pl.ds(c * _CHK, _CHK), :]
        m = lax.dot_general(xb, wc, (((1,), (1,)), ((), ())),
                            preferred_element_type=jnp.float32)
        chunk = sx[:, None] + sw[c * _CHK:(c + 1) * _CHK][None, :] - 2.0 * m
        mc = jnp.min(chunk, axis=1)
        ic = jnp.argmin(chunk, axis=1).astype(jnp.int32) + c * _CHK
        take = (mc < rv) | ((mc == rv) & (ic < idx))
        rv = jnp.where(take, mc, rv).astype(jnp.bfloat16).astype(jnp.float32)
        idx = jnp.where(take, ic, idx)
    idx_ref[0, 0, :] = idx
    acc_ref[...] += jnp.sum(rv).reshape(1, 1)

    @pl.when(i == n_steps - 1)
    def _fin():
        lsum_ref[...] = acc_ref[...]


def _tc_stage(x, w):
    n, d_dim = x.shape
    k = w.shape[0]
    steps = n // _BLK
    return pl.pallas_call(
        functools.partial(_tc_body, steps),
        grid=(steps,),
        in_specs=[
            pl.BlockSpec((_BLK, d_dim), lambda i: (i, 0)),
            pl.BlockSpec((k, d_dim), lambda i: (0, 0)),
        ],
        out_specs=[
            pl.BlockSpec((1, 1, _BLK), lambda i: (i, 0, 0)),
            pl.BlockSpec((1, 1), lambda i: (0, 0)),
        ],
        out_shape=[
            jax.ShapeDtypeStruct((steps, 1, _BLK), jnp.int32),
            jax.ShapeDtypeStruct((1, 1), jnp.float32),
        ],
        scratch_shapes=[
            pltpu.VMEM((1, 1), jnp.float32),
        ],
    )(x, w)


def _sc_stage(w, idx):
    n = idx.shape[0]
    k, d_dim = w.shape
    info = plsc.get_sparse_core_info()
    nw = info.num_cores * info.num_subcores
    bpw = n // nw
    mesh = plsc.VectorSubcoreMesh(core_axis_name="c", subcore_axis_name="s")

    @functools.partial(
        pl.kernel,
        mesh=mesh,
        compiler_params=pltpu.CompilerParams(use_tc_tiling_on_sc=False,
                                             needs_layout_passes=False),
        out_type=[
            jax.ShapeDtypeStruct((n, d_dim), jnp.float32),
            jax.ShapeDtypeStruct((nw, k), jnp.float32),
        ],
        scratch_types=[
            pltpu.VMEM((bpw,), jnp.int32),
            pltpu.VMEM((bpw, d_dim), jnp.float32),
            pltpu.VMEM((k,), jnp.float32),
            pltpu.SemaphoreType.DMA,
        ],
    )
    def gather_hist_kernel(w_hbm, idx_hbm, out_hbm, cnt_hbm,
                           idx_v, rows_v, cnt_v, sem):
        wid = lax.axis_index("s") * info.num_cores + lax.axis_index("c")
        base = wid * bpw
        pltpu.sync_copy(idx_hbm.at[pl.ds(base, bpw)], idx_v)
        gather = pltpu.async_copy(w_hbm.at[idx_v], rows_v, sem)

        zero = jnp.zeros((16,), jnp.float32)

        def zstep(j, carry):
            cnt_v[pl.ds(j * 16, 16)] = zero
            return carry

        lax.fori_loop(0, k // 16, zstep, 0)

        ones = jnp.ones((16,), jnp.float32)

        def hstep(j, carry):
            iv = idx_v[pl.ds(j * 16, 16)]
            plsc.addupdate_scatter(cnt_v, [iv], ones)
            return carry

        lax.fori_loop(0, bpw // 16, hstep, 0)
        pltpu.sync_copy(cnt_v, cnt_hbm.at[wid])

        gather.wait()
        pltpu.sync_copy(rows_v, out_hbm.at[pl.ds(base, bpw)])

    return gather_hist_kernel(w, idx)


def _fin_body(n_rows, d_dim, cnt_ref, lsum_ref, loss_ref, perp_ref):
    counts = jnp.sum(cnt_ref[...], axis=0)
    avg = counts * (1.0 / n_rows)
    ent = jnp.sum(avg * jnp.log(avg + 1e-10))
    perp_ref[...] = jnp.exp(-ent).reshape(1, 1)
    loss_ref[...] = lsum_ref[...] * (0.25 / (n_rows * d_dim))


def _fin_stage(cnt, lsum, n_rows, d_dim):
    nw, k = cnt.shape
    return pl.pallas_call(
        functools.partial(_fin_body, n_rows, d_dim),
        in_specs=[
            pl.BlockSpec((nw, k), lambda: (0, 0)),
            pl.BlockSpec((1, 1), lambda: (0, 0)),
        ],
        out_specs=[
            pl.BlockSpec((1, 1), lambda: (0, 0)),
            pl.BlockSpec((1, 1), lambda: (0, 0)),
        ],
        out_shape=[
            jax.ShapeDtypeStruct((1, 1), jnp.float32),
            jax.ShapeDtypeStruct((1, 1), jnp.float32),
        ],
    )(cnt, lsum)


def kernel(inputs, w):
    d_dim = inputs.shape[-1]
    x = inputs.reshape(-1, d_dim)
    n = x.shape[0]
    idx3, lsum = _tc_stage(x, w)
    idx = idx3.reshape(-1)
    q, cnt = _sc_stage(w, idx)
    loss, perp = _fin_stage(cnt, lsum, n, d_dim)
    quantized = q.reshape(inputs.shape)
    quantized_st = inputs + (quantized - inputs)
    return (loss[0, 0], quantized_st, perp[0, 0], idx[:, None])


# speedup vs baseline: 1.6367x; 1.4852x over previous
import functools

import jax
import jax.numpy as jnp
from jax import lax
from jax.experimental import pallas as pl
from jax.experimental.pallas import tpu as pltpu
from jax.experimental.pallas import tpu_sc as plsc

_BLK = 512
_CHK = 2048


def _tc_body(n_steps, x_ref, w_ref, idx_ref, lsum_ref, acc_ref):
    i = pl.program_id(0)
    k = w_ref.shape[0]

    @pl.when(i == 0)
    def _init():
        acc_ref[...] = jnp.zeros_like(acc_ref)

    x = x_ref[...]
    w = w_ref[...]
    sx = jnp.sum(x * x, axis=1)
    sw = jnp.sum(w * w, axis=1)
    xb = x.astype(jnp.bfloat16)

    rv = jnp.full((x.shape[0],), jnp.inf, jnp.float32)
    idx = jnp.zeros((x.shape[0],), jnp.int32)
    for c in range(k // _CHK):
        wc = w_ref[pl.ds(c * _CHK, _CHK), :]
        m = lax.dot_general(xb, wc, (((1,), (1,)), ((), ())),
                            preferred_element_type=jnp.float32)
        chunk = sx[:, None] + sw[c * _CHK:(c + 1) * _CHK][None, :] - 2.0 * m
        mc = jnp.min(chunk, axis=1)
        ic = jnp.argmin(chunk, axis=1).astype(jnp.int32) + c * _CHK
        take = (mc < rv) | ((mc == rv) & (ic < idx))
        rv = jnp.where(take, mc, rv).astype(jnp.bfloat16).astype(jnp.float32)
        idx = jnp.where(take, ic, idx)
    idx_ref[0, 0, :] = idx
    acc_ref[...] += jnp.sum(rv).reshape(1, 1)

    @pl.when(i == n_steps - 1)
    def _fin():
        lsum_ref[...] = acc_ref[...]


def _tc_stage(x, w):
    n, d_dim = x.shape
    k = w.shape[0]
    steps = n // _BLK
    return pl.pallas_call(
        functools.partial(_tc_body, steps),
        grid=(steps,),
        in_specs=[
            pl.BlockSpec((_BLK, d_dim), lambda i: (i, 0)),
            pl.BlockSpec((k, d_dim), lambda i: (0, 0)),
        ],
        out_specs=[
            pl.BlockSpec((1, 1, _BLK), lambda i: (i, 0, 0)),
            pl.BlockSpec((1, 1), lambda i: (0, 0)),
        ],
        out_shape=[
            jax.ShapeDtypeStruct((steps, 1, _BLK), jnp.int32),
            jax.ShapeDtypeStruct((1, 1), jnp.float32),
        ],
        scratch_shapes=[
            pltpu.VMEM((1, 1), jnp.float32),
        ],
    )(x, w)


def _sc_stage(w, idx):
    n = idx.shape[0]
    k, d_dim = w.shape
    info = plsc.get_sparse_core_info()
    nw = info.num_cores * info.num_subcores
    bpw = n // nw
    mesh = plsc.VectorSubcoreMesh(core_axis_name="c", subcore_axis_name="s")

    @functools.partial(
        pl.kernel,
        mesh=mesh,
        compiler_params=pltpu.CompilerParams(use_tc_tiling_on_sc=False,
                                             needs_layout_passes=False),
        out_type=[
            jax.ShapeDtypeStruct((n, d_dim), jnp.float32),
            jax.ShapeDtypeStruct((nw, k), jnp.float32),
        ],
        scratch_types=[
            pltpu.VMEM((bpw,), jnp.int32),
            pltpu.VMEM((bpw, d_dim), jnp.float32),
            pltpu.VMEM((k,), jnp.float32),
            pltpu.SemaphoreType.DMA,
        ],
    )
    def gather_hist_kernel(w_hbm, idx_hbm, out_hbm, cnt_hbm,
                           idx_v, rows_v, cnt_v, sem):
        wid = lax.axis_index("s") * info.num_cores + lax.axis_index("c")
        base = wid * bpw
        pltpu.sync_copy(idx_hbm.at[pl.ds(base, bpw)], idx_v)
        gather = pltpu.async_copy(w_hbm.at[idx_v], rows_v, sem)

        zero = jnp.zeros((16,), jnp.float32)

        def zstep(j, carry):
            cnt_v[pl.ds(j * 16, 16)] = zero
            return carry

        lax.fori_loop(0, k // 16, zstep, 0)

        ones = jnp.ones((16,), jnp.float32)

        def hstep(j, carry):
            iv = idx_v[pl.ds(j * 16, 16)]
            plsc.addupdate_scatter(cnt_v, [iv], ones)
            return carry

        lax.fori_loop(0, bpw // 16, hstep, 0)
        pltpu.sync_copy(cnt_v, cnt_hbm.at[wid])

        gather.wait()
        pltpu.sync_copy(rows_v, out_hbm.at[pl.ds(base, bpw)])

    return gather_hist_kernel(w, idx)


def _fin_body(n_rows, d_dim, cnt_ref, lsum_ref, loss_ref, perp_ref):
    counts = jnp.sum(cnt_ref[...], axis=0)
    avg = counts * (1.0 / n_rows)
    ent = jnp.sum(avg * jnp.log(avg + 1e-10))
    perp_ref[...] = jnp.exp(-ent).reshape(1, 1)
    loss_ref[...] = lsum_ref[...] * (0.25 / (n_rows * d_dim))


def _fin_stage(cnt, lsum, n_rows, d_dim):
    nw, k = cnt.shape
    return pl.pallas_call(
        functools.partial(_fin_body, n_rows, d_dim),
        in_specs=[
            pl.BlockSpec((nw, k), lambda: (0, 0)),
            pl.BlockSpec((1, 1), lambda: (0, 0)),
        ],
        out_specs=[
            pl.BlockSpec((1, 1), lambda: (0, 0)),
            pl.BlockSpec((1, 1), lambda: (0, 0)),
        ],
        out_shape=[
            jax.ShapeDtypeStruct((1, 1), jnp.float32),
            jax.ShapeDtypeStruct((1, 1), jnp.float32),
        ],
    )(cnt, lsum)


def kernel(inputs, w):
    d_dim = inputs.shape[-1]
    x = inputs.reshape(-1, d_dim)
    n = x.shape[0]
    idx3, lsum = _tc_stage(x, w)
    idx = idx3.reshape(-1)
    q, cnt = _sc_stage(w, idx)
    loss, perp = _fin_stage(cnt, lsum, n, d_dim)
    quantized = q.reshape(inputs.shape)
    quantized_st = inputs + (quantized - inputs)
    return (loss[0, 0], quantized_st, perp[0, 0], idx[:, None])


# speedup vs baseline: 1.8539x; 1.1327x over previous
import functools

import jax
import jax.numpy as jnp
from jax import lax
from jax.experimental import pallas as pl
from jax.experimental.pallas import tpu as pltpu
from jax.experimental.pallas import tpu_sc as plsc

_BLK = 1024
_CHK = 2048


def _tc_body(n_steps, x_ref, w_ref, idx_ref, lsum_ref, acc_ref):
    i = pl.program_id(0)
    k = w_ref.shape[0]

    @pl.when(i == 0)
    def _init():
        acc_ref[...] = jnp.zeros_like(acc_ref)

    x = x_ref[...]
    w = w_ref[...]
    sx = jnp.sum(x * x, axis=1)
    sw = jnp.sum(w * w, axis=1)
    xb = x.astype(jnp.bfloat16)

    rv = jnp.full((x.shape[0],), jnp.inf, jnp.float32)
    idx = jnp.zeros((x.shape[0],), jnp.int32)
    for c in range(k // _CHK):
        wc = w_ref[pl.ds(c * _CHK, _CHK), :]
        m = lax.dot_general(xb, wc, (((1,), (1,)), ((), ())),
                            preferred_element_type=jnp.float32)
        chunk = sx[:, None] + sw[c * _CHK:(c + 1) * _CHK][None, :] - 2.0 * m
        mc = jnp.min(chunk, axis=1)
        ic = jnp.argmin(chunk, axis=1).astype(jnp.int32) + c * _CHK
        take = (mc < rv) | ((mc == rv) & (ic < idx))
        rv = jnp.where(take, mc, rv).astype(jnp.bfloat16).astype(jnp.float32)
        idx = jnp.where(take, ic, idx)
    idx_ref[0, 0, :] = idx
    acc_ref[...] += jnp.sum(rv).reshape(1, 1)

    @pl.when(i == n_steps - 1)
    def _fin():
        lsum_ref[...] = acc_ref[...]


def _tc_stage(x, w):
    n, d_dim = x.shape
    k = w.shape[0]
    steps = n // _BLK
    return pl.pallas_call(
        functools.partial(_tc_body, steps),
        grid=(steps,),
        in_specs=[
            pl.BlockSpec((_BLK, d_dim), lambda i: (i, 0)),
            pl.BlockSpec((k, d_dim), lambda i: (0, 0)),
        ],
        out_specs=[
            pl.BlockSpec((1, 1, _BLK), lambda i: (i, 0, 0)),
            pl.BlockSpec((1, 1), lambda i: (0, 0)),
        ],
        out_shape=[
            jax.ShapeDtypeStruct((steps, 1, _BLK), jnp.int32),
            jax.ShapeDtypeStruct((1, 1), jnp.float32),
        ],
        scratch_shapes=[
            pltpu.VMEM((1, 1), jnp.float32),
        ],
    )(x, w)


def _sc_stage(w, idx):
    n = idx.shape[0]
    k, d_dim = w.shape
    info = plsc.get_sparse_core_info()
    nw = info.num_cores * info.num_subcores
    bpw = n // nw
    mesh = plsc.VectorSubcoreMesh(core_axis_name="c", subcore_axis_name="s")

    @functools.partial(
        pl.kernel,
        mesh=mesh,
        compiler_params=pltpu.CompilerParams(use_tc_tiling_on_sc=False,
                                             needs_layout_passes=False),
        out_type=[
            jax.ShapeDtypeStruct((n, d_dim), jnp.float32),
            jax.ShapeDtypeStruct((nw, k), jnp.float32),
        ],
        scratch_types=[
            pltpu.VMEM((bpw,), jnp.int32),
            pltpu.VMEM((bpw, d_dim), jnp.float32),
            pltpu.VMEM((k,), jnp.float32),
            pltpu.SemaphoreType.DMA,
        ],
    )
    def gather_hist_kernel(w_hbm, idx_hbm, out_hbm, cnt_hbm,
                           idx_v, rows_v, cnt_v, sem):
        wid = lax.axis_index("s") * info.num_cores + lax.axis_index("c")
        base = wid * bpw
        pltpu.sync_copy(idx_hbm.at[pl.ds(base, bpw)], idx_v)
        gather = pltpu.async_copy(w_hbm.at[idx_v], rows_v, sem)

        zero = jnp.zeros((16,), jnp.float32)

        def zstep(j, carry):
            cnt_v[pl.ds(j * 16, 16)] = zero
            return carry

        lax.fori_loop(0, k // 16, zstep, 0)

        ones = jnp.ones((16,), jnp.float32)

        def hstep(j, carry):
            iv = idx_v[pl.ds(j * 16, 16)]
            plsc.addupdate_scatter(cnt_v, [iv], ones)
            return carry

        lax.fori_loop(0, bpw // 16, hstep, 0)
        pltpu.sync_copy(cnt_v, cnt_hbm.at[wid])

        gather.wait()
        pltpu.sync_copy(rows_v, out_hbm.at[pl.ds(base, bpw)])

    return gather_hist_kernel(w, idx)


def _fin_body(n_rows, d_dim, cnt_ref, lsum_ref, loss_ref, perp_ref):
    counts = jnp.sum(cnt_ref[...], axis=0)
    avg = counts * (1.0 / n_rows)
    ent = jnp.sum(avg * jnp.log(avg + 1e-10))
    perp_ref[...] = jnp.exp(-ent).reshape(1, 1)
    loss_ref[...] = lsum_ref[...] * (0.25 / (n_rows * d_dim))


def _fin_stage(cnt, lsum, n_rows, d_dim):
    nw, k = cnt.shape
    return pl.pallas_call(
        functools.partial(_fin_body, n_rows, d_dim),
        in_specs=[
            pl.BlockSpec((nw, k), lambda: (0, 0)),
            pl.BlockSpec((1, 1), lambda: (0, 0)),
        ],
        out_specs=[
            pl.BlockSpec((1, 1), lambda: (0, 0)),
            pl.BlockSpec((1, 1), lambda: (0, 0)),
        ],
        out_shape=[
            jax.ShapeDtypeStruct((1, 1), jnp.float32),
            jax.ShapeDtypeStruct((1, 1), jnp.float32),
        ],
    )(cnt, lsum)


def kernel(inputs, w):
    d_dim = inputs.shape[-1]
    x = inputs.reshape(-1, d_dim)
    n = x.shape[0]
    idx3, lsum = _tc_stage(x, w)
    idx = idx3.reshape(-1)
    q, cnt = _sc_stage(w, idx)
    loss, perp = _fin_stage(cnt, lsum, n, d_dim)
    quantized = q.reshape(inputs.shape)
    quantized_st = inputs + (quantized - inputs)
    return (loss[0, 0], quantized_st, perp[0, 0], idx[:, None])


# speedup vs baseline: 1.8744x; 1.0111x over previous
import functools

import jax
import jax.numpy as jnp
from jax import lax
from jax.experimental import pallas as pl
from jax.experimental.pallas import tpu as pltpu
from jax.experimental.pallas import tpu_sc as plsc

_BLK = 2304
_CHK = 2048


def _tc_body(n_steps, x_ref, w_ref, idx_ref, lsum_ref, acc_ref):
    i = pl.program_id(0)
    k = w_ref.shape[0]

    @pl.when(i == 0)
    def _init():
        acc_ref[...] = jnp.zeros_like(acc_ref)

    x = x_ref[...]
    w = w_ref[...]
    sx = jnp.sum(x * x, axis=1)
    sw = jnp.sum(w * w, axis=1)
    xb = x.astype(jnp.bfloat16)

    rv = jnp.full((x.shape[0],), jnp.inf, jnp.float32)
    idx = jnp.zeros((x.shape[0],), jnp.int32)
    for c in range(k // _CHK):
        wc = w_ref[pl.ds(c * _CHK, _CHK), :]
        m = lax.dot_general(xb, wc, (((1,), (1,)), ((), ())),
                            preferred_element_type=jnp.float32)
        chunk = sx[:, None] + sw[c * _CHK:(c + 1) * _CHK][None, :] - 2.0 * m
        mc = jnp.min(chunk, axis=1)
        ic = jnp.argmin(chunk, axis=1).astype(jnp.int32) + c * _CHK
        take = (mc < rv) | ((mc == rv) & (ic < idx))
        rv = jnp.where(take, mc, rv).astype(jnp.bfloat16).astype(jnp.float32)
        idx = jnp.where(take, ic, idx)
    idx_ref[0, 0, :] = idx
    acc_ref[...] += jnp.sum(rv).reshape(1, 1)

    @pl.when(i == n_steps - 1)
    def _fin():
        lsum_ref[...] = acc_ref[...]


def _tc_stage(x, w):
    n, d_dim = x.shape
    k = w.shape[0]
    steps = n // _BLK
    return pl.pallas_call(
        functools.partial(_tc_body, steps),
        grid=(steps,),
        in_specs=[
            pl.BlockSpec((_BLK, d_dim), lambda i: (i, 0)),
            pl.BlockSpec((k, d_dim), lambda i: (0, 0)),
        ],
        out_specs=[
            pl.BlockSpec((1, 1, _BLK), lambda i: (i, 0, 0)),
            pl.BlockSpec((1, 1), lambda i: (0, 0)),
        ],
        out_shape=[
            jax.ShapeDtypeStruct((steps, 1, _BLK), jnp.int32),
            jax.ShapeDtypeStruct((1, 1), jnp.float32),
        ],
        scratch_shapes=[
            pltpu.VMEM((1, 1), jnp.float32),
        ],
    )(x, w)


def _sc_stage(w, idx):
    n = idx.shape[0]
    k, d_dim = w.shape
    info = plsc.get_sparse_core_info()
    nw = info.num_cores * info.num_subcores
    bpw = n // nw
    mesh = plsc.VectorSubcoreMesh(core_axis_name="c", subcore_axis_name="s")

    @functools.partial(
        pl.kernel,
        mesh=mesh,
        compiler_params=pltpu.CompilerParams(use_tc_tiling_on_sc=False,
                                             needs_layout_passes=False),
        out_type=[
            jax.ShapeDtypeStruct((n, d_dim), jnp.float32),
            jax.ShapeDtypeStruct((nw, k), jnp.float32),
        ],
        scratch_types=[
            pltpu.VMEM((bpw,), jnp.int32),
            pltpu.VMEM((bpw, d_dim), jnp.float32),
            pltpu.VMEM((k,), jnp.float32),
            pltpu.SemaphoreType.DMA,
        ],
    )
    def gather_hist_kernel(w_hbm, idx_hbm, out_hbm, cnt_hbm,
                           idx_v, rows_v, cnt_v, sem):
        wid = lax.axis_index("s") * info.num_cores + lax.axis_index("c")
        base = wid * bpw
        pltpu.sync_copy(idx_hbm.at[pl.ds(base, bpw)], idx_v)
        gather = pltpu.async_copy(w_hbm.at[idx_v], rows_v, sem)

        zero = jnp.zeros((16,), jnp.float32)

        def zstep(j, carry):
            cnt_v[pl.ds(j * 16, 16)] = zero
            return carry

        lax.fori_loop(0, k // 16, zstep, 0)

        ones = jnp.ones((16,), jnp.float32)

        def hstep(j, carry):
            iv = idx_v[pl.ds(j * 16, 16)]
            plsc.addupdate_scatter(cnt_v, [iv], ones)
            return carry

        lax.fori_loop(0, bpw // 16, hstep, 0)
        pltpu.sync_copy(cnt_v, cnt_hbm.at[wid])

        gather.wait()
        pltpu.sync_copy(rows_v, out_hbm.at[pl.ds(base, bpw)])

    return gather_hist_kernel(w, idx)


def _fin_body(n_rows, d_dim, cnt_ref, lsum_ref, loss_ref, perp_ref):
    counts = jnp.sum(cnt_ref[...], axis=0)
    avg = counts * (1.0 / n_rows)
    ent = jnp.sum(avg * jnp.log(avg + 1e-10))
    perp_ref[...] = jnp.exp(-ent).reshape(1, 1)
    loss_ref[...] = lsum_ref[...] * (0.25 / (n_rows * d_dim))


def _fin_stage(cnt, lsum, n_rows, d_dim):
    nw, k = cnt.shape
    return pl.pallas_call(
        functools.partial(_fin_body, n_rows, d_dim),
        in_specs=[
            pl.BlockSpec((nw, k), lambda: (0, 0)),
            pl.BlockSpec((1, 1), lambda: (0, 0)),
        ],
        out_specs=[
            pl.BlockSpec((1, 1), lambda: (0, 0)),
            pl.BlockSpec((1, 1), lambda: (0, 0)),
        ],
        out_shape=[
            jax.ShapeDtypeStruct((1, 1), jnp.float32),
            jax.ShapeDtypeStruct((1, 1), jnp.float32),
        ],
    )(cnt, lsum)


def kernel(inputs, w):
    d_dim = inputs.shape[-1]
    x = inputs.reshape(-1, d_dim)
    n = x.shape[0]
    idx3, lsum = _tc_stage(x, w)
    idx = idx3.reshape(-1)
    q, cnt = _sc_stage(w, idx)
    loss, perp = _fin_stage(cnt, lsum, n, d_dim)
    quantized = q.reshape(inputs.shape)
    quantized_st = inputs + (quantized - inputs)
    return (loss[0, 0], quantized_st, perp[0, 0], idx[:, None])
